# baseline (speedup 1.0000x reference)
import functools

import jax
import jax.numpy as jnp
from jax.experimental import pallas as pl
from jax.experimental.pallas import tpu as pltpu

_EPS = 1e-5
_NEG = -1e30


def _body(x_ref, mpt_ref, a_in_ref,
          wpreT_ref, gpre_ref, bpre_ref,
          w1T_ref, g1_ref, b1_ref,
          w2T_ref, g2_ref, b2_ref,
          wo1T_ref, bo1_ref, wo2T_ref, bo2_ref,
          out_ref, a_out_ref, stat, segmax,
          *, G, NPAD, H):
    ph = pl.program_id(0)
    i = pl.program_id(1)
    nb = pl.num_programs(1)
    R = G * NPAD

    @pl.when(jnp.logical_and(ph == 0, i == 0))
    def _init():
        stat[...] = jnp.zeros_like(stat)

    def accum(a, am):
        stat[0:1] += jnp.sum(am, axis=0, keepdims=True)
        stat[1:2] += jnp.sum(am * a, axis=0, keepdims=True)

    def finalize(g_ref, b_ref, srow):
        cnt = jnp.maximum(stat[14:15], 1.0)
        mean = stat[0:1] / cnt
        var = stat[1:2] / cnt - mean * mean
        s = g_ref[...] / jnp.sqrt(var + _EPS)
        t = b_ref[...] - mean * s
        stat[srow:srow + 1] = s
        stat[srow + 1:srow + 2] = t
        stat[0:2] = jnp.zeros((2, H), jnp.float32)

    def a1_fn():
        return jnp.dot(x_ref[...], wpreT_ref[...],
                       preferred_element_type=jnp.float32)

    @pl.when(ph == 0)
    def _p0():
        m = mpt_ref[...].astype(jnp.float32)
        a1 = a1_fn()
        accum(a1, a1 * m)
        stat[14:15] += jnp.sum(m)

    @pl.when(jnp.logical_and(ph == 0, i == nb - 1))
    def _f0():
        finalize(gpre_ref, bpre_ref, 8)

    @pl.when(ph == 1)
    def _p1():
        m = mpt_ref[...].astype(jnp.float32)
        a1 = a1_fn()
        feat = jnp.maximum(a1 * stat[8:9] + stat[9:10], 0.0) * m
        pooled = jnp.max(feat.reshape(G, NPAD, H), axis=1)
        pc = jnp.dot(pooled, w1T_ref[H:2 * H, :],
                     preferred_element_type=jnp.float32)
        pc3 = jnp.broadcast_to(pc[:, None, :], (G, NPAD, H)).reshape(R, H)
        a2 = jnp.dot(feat, w1T_ref[0:H, :],
                     preferred_element_type=jnp.float32) + pc3
        accum(a2, a2 * m)
        a_out_ref[...] = a2.astype(jnp.bfloat16)

    @pl.when(jnp.logical_and(ph == 1, i == nb - 1))
    def _f1():
        finalize(g1_ref, b1_ref, 10)

    @pl.when(ph == 2)
    def _p2():
        m = mpt_ref[...].astype(jnp.float32)
        a2 = a_in_ref[...].astype(jnp.float32)
        h2 = jnp.maximum(a2 * stat[10:11] + stat[11:12], 0.0) * m
        a3 = jnp.dot(h2, w2T_ref[...], preferred_element_type=jnp.float32)
        accum(a3, a3 * m)
        z = jnp.where(m > 0.0, a3, _NEG)
        segmax[pl.ds(i * G, G), :] = jnp.max(z.reshape(G, NPAD, H), axis=1)

    @pl.when(jnp.logical_and(ph == 2, i == nb - 1))
    def _f2():
        finalize(g2_ref, b2_ref, 12)

    @pl.when(ph == 3)
    def _p3():
        sm = segmax[pl.ds(i * G, G), :]
        buf = jnp.maximum(sm * stat[12:13] + stat[13:14], 0.0)
        o1 = jnp.maximum(
            jnp.dot(buf, wo1T_ref[...], preferred_element_type=jnp.float32)
            + bo1_ref[...], 0.0)
        o = jnp.dot(o1, wo2T_ref[...],
                    preferred_element_type=jnp.float32) + bo2_ref[...]
        valid = sm[:, 0:1] > (0.5 * _NEG)
        out_ref[...] = o * valid.astype(jnp.float32)


def kernel(polylines, polylines_mask, W_pre, g_pre, b_pre,
           W1, g1, b1, W2, g2, b2, Wo1, bo1, Wo2, bo2):
    B, P, N, C = polylines.shape
    H = W_pre.shape[0]
    O = Wo2.shape[0]
    BP = B * P
    NPAD = ((N + 7) // 8) * 8
    G = 128
    NB = BP // G
    R = G * NPAD

    xp = jnp.pad(polylines.reshape(BP, N, C),
                 ((0, 0), (0, NPAD - N), (0, 0))).reshape(BP * NPAD, C)
    mpt = jnp.pad(polylines_mask.astype(jnp.bfloat16).reshape(BP, N),
                  ((0, 0), (0, NPAD - N))).reshape(BP * NPAD, 1)
    a_buf = jnp.zeros((BP * NPAD, H), jnp.bfloat16)

    row = lambda v: v.reshape(1, -1)

    def x_idx(ph, i):
        return (jnp.where(ph < 2, i, 0), 0)

    def pts_idx(ph, i):
        return (jnp.where(ph < 3, i, 0), 0)

    def a_in_idx(ph, i):
        return (jnp.where(ph == 2, i, 1), 0)

    def a_out_idx(ph, i):
        return (jnp.where(ph == 1, i, 0), 0)

    def poly_idx(ph, i):
        return (jnp.where(ph == 3, i, 0), 0)

    full = lambda shape: pl.BlockSpec(shape, lambda ph, i: (0, 0))

    body = functools.partial(_body, G=G, NPAD=NPAD, H=H)

    out, _ = pl.pallas_call(
        body,
        grid=(4, NB),
        in_specs=[
            pl.BlockSpec((R, C), x_idx),
            pl.BlockSpec((R, 1), pts_idx),
            pl.BlockSpec((R, H), a_in_idx),
            full((C, H)), full((1, H)), full((1, H)),
            full((2 * H, H)), full((1, H)), full((1, H)),
            full((H, H)), full((1, H)), full((1, H)),
            full((H, H)), full((1, H)), full((H, O)), full((1, O)),
        ],
        out_specs=[
            pl.BlockSpec((G, O), poly_idx),
            pl.BlockSpec((R, H), a_out_idx),
        ],
        out_shape=[
            jax.ShapeDtypeStruct((BP, O), jnp.float32),
            jax.ShapeDtypeStruct((BP * NPAD, H), jnp.bfloat16),
        ],
        input_output_aliases={2: 1},
        scratch_shapes=[
            pltpu.VMEM((16, H), jnp.float32),
            pltpu.VMEM((BP, H), jnp.float32),
        ],
    )(xp, mpt, a_buf,
      W_pre.T, row(g_pre), row(b_pre),
      W1.T, row(g1), row(b1),
      W2.T, row(g2), row(b2),
      Wo1.T, row(bo1), Wo2.T, row(bo2))
    return out.reshape(B, P, O)


# speedup vs baseline: 1.1491x; 1.1491x over previous
import functools

import jax
import jax.numpy as jnp
from jax.experimental import pallas as pl
from jax.experimental.pallas import tpu as pltpu

_EPS = 1e-5
_NEG = -1e30


def _body(x_ref, mpt_ref, a_in_ref,
          wpreT_ref, gpre_ref, bpre_ref,
          w1T_ref, g1_ref, b1_ref,
          w2T_ref, g2_ref, b2_ref,
          wo1T_ref, bo1_ref, wo2T_ref, bo2_ref,
          out_ref, a_out_ref, stat, segmax,
          *, G, NPAD, H):
    ph = pl.program_id(0)
    i = pl.program_id(1)
    nb = pl.num_programs(1)
    R = G * NPAD

    @pl.when(jnp.logical_and(ph == 0, i == 0))
    def _init():
        stat[...] = jnp.zeros_like(stat)

    def accum(a, am):
        stat[0:1] += jnp.sum(am, axis=0, keepdims=True)
        stat[1:2] += jnp.sum(am * a, axis=0, keepdims=True)

    def finalize(g_ref, b_ref, srow):
        cnt = jnp.maximum(stat[14:15], 1.0)
        mean = stat[0:1] / cnt
        var = stat[1:2] / cnt - mean * mean
        s = g_ref[...] / jnp.sqrt(var + _EPS)
        t = b_ref[...] - mean * s
        stat[srow:srow + 1] = s
        stat[srow + 1:srow + 2] = t
        stat[0:2] = jnp.zeros((2, H), jnp.float32)

    def a1_fn():
        return jnp.dot(x_ref[...], wpreT_ref[...],
                       preferred_element_type=jnp.float32)

    @pl.when(ph == 0)
    def _p0():
        m = mpt_ref[...].astype(jnp.float32)
        a1 = a1_fn()
        accum(a1, a1 * m)
        stat[14:15] += jnp.sum(m)

    @pl.when(jnp.logical_and(ph == 0, i == nb - 1))
    def _f0():
        finalize(gpre_ref, bpre_ref, 8)

    @pl.when(ph == 1)
    def _p1():
        m = mpt_ref[...].astype(jnp.float32)
        a1 = a1_fn()
        feat = jnp.maximum(a1 * stat[8:9] + stat[9:10], 0.0) * m
        pooled = jnp.max(feat.reshape(G, NPAD, H), axis=1)
        pc = jnp.dot(pooled, w1T_ref[H:2 * H, :],
                     preferred_element_type=jnp.float32)
        pc3 = jnp.broadcast_to(pc[:, None, :], (G, NPAD, H)).reshape(R, H)
        a2 = jnp.dot(feat, w1T_ref[0:H, :],
                     preferred_element_type=jnp.float32) + pc3
        accum(a2, a2 * m)
        a_out_ref[...] = a2.astype(jnp.bfloat16)

    @pl.when(jnp.logical_and(ph == 1, i == nb - 1))
    def _f1():
        finalize(g1_ref, b1_ref, 10)

    @pl.when(ph == 2)
    def _p2():
        m = mpt_ref[...].astype(jnp.float32)
        a2 = a_in_ref[...].astype(jnp.float32)
        h2 = jnp.maximum(a2 * stat[10:11] + stat[11:12], 0.0) * m
        a3 = jnp.dot(h2, w2T_ref[...], preferred_element_type=jnp.float32)
        accum(a3, a3 * m)
        z = jnp.where(m > 0.0, a3, _NEG)
        segmax[pl.ds(i * G, G), :] = jnp.max(z.reshape(G, NPAD, H), axis=1)

    @pl.when(jnp.logical_and(ph == 2, i == nb - 1))
    def _f2():
        finalize(g2_ref, b2_ref, 12)

    @pl.when(ph == 3)
    def _p3():
        sm = segmax[pl.ds(i * G, G), :]
        buf = jnp.maximum(sm * stat[12:13] + stat[13:14], 0.0)
        o1 = jnp.maximum(
            jnp.dot(buf, wo1T_ref[...], preferred_element_type=jnp.float32)
            + bo1_ref[...], 0.0)
        o = jnp.dot(o1, wo2T_ref[...],
                    preferred_element_type=jnp.float32) + bo2_ref[...]
        valid = sm[:, 0:1] > (0.5 * _NEG)
        out_ref[...] = o * valid.astype(jnp.float32)


def kernel(polylines, polylines_mask, W_pre, g_pre, b_pre,
           W1, g1, b1, W2, g2, b2, Wo1, bo1, Wo2, bo2):
    B, P, N, C = polylines.shape
    H = W_pre.shape[0]
    O = Wo2.shape[0]
    BP = B * P
    NPAD = ((N + 7) // 8) * 8
    G = 256
    NB = BP // G
    R = G * NPAD

    xp = jnp.pad(polylines.reshape(BP, N, C),
                 ((0, 0), (0, NPAD - N), (0, 0))).reshape(BP * NPAD, C)
    mpt = jnp.pad(polylines_mask.astype(jnp.bfloat16).reshape(BP, N),
                  ((0, 0), (0, NPAD - N))).reshape(BP * NPAD, 1)
    a_buf = jnp.zeros((BP * NPAD, H), jnp.bfloat16)

    row = lambda v: v.reshape(1, -1)

    def x_idx(ph, i):
        return (jnp.where(ph < 2, i, 0), 0)

    def pts_idx(ph, i):
        return (jnp.where(ph < 3, i, 0), 0)

    def a_in_idx(ph, i):
        return (jnp.where(ph == 2, i, 1), 0)

    def a_out_idx(ph, i):
        return (jnp.where(ph == 1, i, 0), 0)

    def poly_idx(ph, i):
        return (jnp.where(ph == 3, i, 0), 0)

    full = lambda shape: pl.BlockSpec(shape, lambda ph, i: (0, 0))

    body = functools.partial(_body, G=G, NPAD=NPAD, H=H)

    out, _ = pl.pallas_call(
        body,
        grid=(4, NB),
        in_specs=[
            pl.BlockSpec((R, C), x_idx),
            pl.BlockSpec((R, 1), pts_idx),
            pl.BlockSpec((R, H), a_in_idx),
            full((C, H)), full((1, H)), full((1, H)),
            full((2 * H, H)), full((1, H)), full((1, H)),
            full((H, H)), full((1, H)), full((1, H)),
            full((H, H)), full((1, H)), full((H, O)), full((1, O)),
        ],
        out_specs=[
            pl.BlockSpec((G, O), poly_idx),
            pl.BlockSpec((R, H), a_out_idx),
        ],
        out_shape=[
            jax.ShapeDtypeStruct((BP, O), jnp.float32),
            jax.ShapeDtypeStruct((BP * NPAD, H), jnp.bfloat16),
        ],
        input_output_aliases={2: 1},
        scratch_shapes=[
            pltpu.VMEM((16, H), jnp.float32),
            pltpu.VMEM((BP, H), jnp.float32),
        ],
    )(xp, mpt, a_buf,
      W_pre.T, row(g_pre), row(b_pre),
      W1.T, row(g1), row(b1),
      W2.T, row(g2), row(b2),
      Wo1.T, row(bo1), Wo2.T, row(bo2))
    return out.reshape(B, P, O)
